# baseline (speedup 1.0000x reference)
import functools

import jax
import jax.numpy as jnp
from jax import lax
from jax.experimental import pallas as pl
from jax.experimental.pallas import tpu as pltpu
from jax.experimental.pallas import tpu_sc as plsc

_EMB_DIM = 128
_D0 = 42
_D1 = 42
_DD = 44
_V0 = 98
_VD = 21
_MAX_DEPTH = 20
_N = 100000
_B = 96
_K = 32
_PAIRS = _K // 2
_RANGE = _B * _K
_NW = 32
_TAILBASE = _NW * _RANGE
_TAIL = 48
_TAIL2 = _N - _TAILBASE - _NW * _TAIL
_V1P = 10112
_SLAB = _V1P // 16


def kernel(x, depth, emb0, emb1, depth_table):
    x0 = x[:, 0].astype(jnp.int32)
    x1 = x[:, 1].astype(jnp.int32)
    dep = depth.astype(jnp.int32)
    e1p = jnp.pad(emb1, ((0, _V1P - emb1.shape[0]),
                         (_D0, _EMB_DIM - _D0 - _D1)))
    e0f = emb0.reshape(-1)
    edf = depth_table.reshape(-1)

    mesh = plsc.VectorSubcoreMesh(core_axis_name="c", subcore_axis_name="s")
    nidx = _RANGE + _TAIL + _TAIL2

    @functools.partial(
        pl.kernel,
        mesh=mesh,
        compiler_params=pltpu.CompilerParams(needs_layout_passes=False),
        out_type=jax.ShapeDtypeStruct((_N, _EMB_DIM), jnp.float32),
        scratch_types=[
            pltpu.VMEM((nidx,), jnp.int32),
            pltpu.VMEM((nidx,), jnp.int32),
            pltpu.VMEM((nidx,), jnp.int32),
            pltpu.VMEM((_V0 * _D0,), jnp.float32),
            pltpu.VMEM((_VD * _DD,), jnp.float32),
            pltpu.VMEM((_B, _EMB_DIM), jnp.float32),
            pltpu.VMEM((_B, _EMB_DIM), jnp.float32),
            pltpu.VMEM_SHARED((_V1P, _EMB_DIM), jnp.float32),
            pltpu.SemaphoreType.DMA,
            pltpu.SemaphoreType.DMA,
            pltpu.SemaphoreType.DMA,
        ],
    )
    def run(x0_hbm, x1_hbm, dep_hbm, e0_hbm, e1_hbm, ed_hbm, out_hbm,
            idx0, idx1, idxd, e0_res, ed_res, rows_a, rows_b, e1_sh,
            gsem, wsem, isem):
        wid = lax.axis_index("s") * 2 + lax.axis_index("c")
        sid = lax.axis_index("s")
        base = wid * _RANGE
        lanes = lax.iota(jnp.int32, 16)

        pltpu.sync_copy(e0_hbm, e0_res)
        pltpu.sync_copy(ed_hbm, ed_res)
        tbase = _TAILBASE + wid * _TAIL
        stage = [
            pltpu.async_copy(e1_hbm.at[pl.ds(sid * _SLAB, _SLAB)],
                             e1_sh.at[pl.ds(sid * _SLAB, _SLAB)], isem),
            pltpu.async_copy(x0_hbm.at[pl.ds(base, _RANGE)],
                             idx0.at[pl.ds(0, _RANGE)], isem),
            pltpu.async_copy(x1_hbm.at[pl.ds(base, _RANGE)],
                             idx1.at[pl.ds(0, _RANGE)], isem),
            pltpu.async_copy(dep_hbm.at[pl.ds(base, _RANGE)],
                             idxd.at[pl.ds(0, _RANGE)], isem),
            pltpu.async_copy(x0_hbm.at[pl.ds(tbase, _TAIL)],
                             idx0.at[pl.ds(_RANGE, _TAIL)], isem),
            pltpu.async_copy(x1_hbm.at[pl.ds(tbase, _TAIL)],
                             idx1.at[pl.ds(_RANGE, _TAIL)], isem),
            pltpu.async_copy(dep_hbm.at[pl.ds(tbase, _TAIL)],
                             idxd.at[pl.ds(_RANGE, _TAIL)], isem),
        ]
        for s in stage:
            s.wait()
        plsc.subcore_barrier()

        def fire_gather(ioff, nrows, buf):
            return pltpu.async_copy(
                e1_sh.at[idx1.at[pl.ds(ioff, nrows)]],
                buf.at[pl.ds(0, nrows)], gsem)

        def wait_gather(nrows, buf):
            pltpu.make_async_copy(
                e1_hbm.at[pl.ds(0, nrows)], buf.at[pl.ds(0, nrows)], gsem).wait()

        def fill(ioff, nrows, buf):
            @plsc.parallel_loop(0, nrows // 16, unroll=1)
            def grp(t):
                r0 = t * 16
                a = idx0[pl.ds(ioff + r0, 16)] * _D0
                d = jnp.minimum(idxd[pl.ds(ioff + r0, 16)], _MAX_DEPTH) * _DD
                for l in range(16):
                    r = r0 + l
                    i0 = a[l]
                    idp = d[l]
                    buf[r, pl.ds(0, 16)] = e0_res[pl.ds(i0, 16)]
                    buf[r, pl.ds(16, 16)] = e0_res[pl.ds(i0 + 16, 16)]
                    buf[r, pl.ds(_D0 - 16, 16)] = e0_res[pl.ds(i0 + _D0 - 16, 16)]
                    buf[r, pl.ds(_D0 + _D1, 16)] = ed_res[pl.ds(idp, 16)]
                    buf[r, pl.ds(_D0 + _D1 + 16, 16)] = ed_res[pl.ds(idp + 16, 16)]
                    buf[r, pl.ds(_EMB_DIM - 16, 16)] = ed_res[pl.ds(idp + _DD - 16, 16)]

        def fire_write(obase, nrows, buf):
            return pltpu.async_copy(
                buf.at[pl.ds(0, nrows)], out_hbm.at[pl.ds(obase, nrows)], wsem)

        def wait_write(nrows, buf):
            pltpu.make_async_copy(
                buf.at[pl.ds(0, nrows)], out_hbm.at[pl.ds(0, nrows)], wsem).wait()

        fire_gather(0, _B, rows_a)
        fire_gather(_B, _B, rows_b)

        def pair(p, _):
            ca = 2 * p * _B
            cb = ca + _B
            wait_gather(_B, rows_a)
            fill(ca, _B, rows_a)
            fire_write(base + ca, _B, rows_a)
            wait_gather(_B, rows_b)

            @pl.when(p < _PAIRS - 1)
            def _():
                wait_write(_B, rows_a)
                fire_gather(cb + _B, _B, rows_a)
            fill(cb, _B, rows_b)
            fire_write(base + cb, _B, rows_b)

            @pl.when(p < _PAIRS - 1)
            def _():
                wait_write(_B, rows_b)
                fire_gather(cb + 2 * _B, _B, rows_b)
            return 0
        lax.fori_loop(0, _PAIRS, pair, 0)

        wait_write(_B, rows_a)
        fire_gather(_RANGE, _TAIL, rows_a)
        wait_gather(_TAIL, rows_a)
        fill(_RANGE, _TAIL, rows_a)
        tdesc = fire_write(tbase, _TAIL, rows_a)
        wait_write(_B, rows_b)
        tdesc.wait()

        @pl.when(wid == _NW - 1)
        def _():
            t2base = _TAILBASE + _NW * _TAIL
            s2 = [pltpu.async_copy(x0_hbm.at[pl.ds(t2base, _TAIL2)],
                                   idx0.at[pl.ds(_RANGE + _TAIL, _TAIL2)], isem),
                  pltpu.async_copy(x1_hbm.at[pl.ds(t2base, _TAIL2)],
                                   idx1.at[pl.ds(_RANGE + _TAIL, _TAIL2)], isem),
                  pltpu.async_copy(dep_hbm.at[pl.ds(t2base, _TAIL2)],
                                   idxd.at[pl.ds(_RANGE + _TAIL, _TAIL2)], isem)]
            for s in s2:
                s.wait()
            for off, cnt in ((0, _B), (_B, _TAIL2 - _B)):
                fire_gather(_RANGE + _TAIL + off, cnt, rows_b)
                wait_gather(cnt, rows_b)
                fill(_RANGE + _TAIL + off, cnt, rows_b)
                fire_write(t2base + off, cnt, rows_b).wait()

    return run(x0, x1, dep, e0f, e1p, edf)


# speedup vs baseline: 1.1520x; 1.1520x over previous
import functools

import jax
import jax.numpy as jnp
from jax import lax
from jax.experimental import pallas as pl
from jax.experimental.pallas import tpu as pltpu
from jax.experimental.pallas import tpu_sc as plsc

_EMB_DIM = 128
_D0 = 42
_D1 = 42
_DD = 44
_V0 = 98
_VD = 21
_MAX_DEPTH = 20
_N = 100000
_B = 128
_K = 24
_PAIRS = _K // 2
_RANGE = _B * _K
_NW = 32
_TAILBASE = _NW * _RANGE
_TAIL = 48
_TAIL2 = _N - _TAILBASE - _NW * _TAIL
_V1P = 10112
_SLAB = _V1P // 16


def kernel(x, depth, emb0, emb1, depth_table):
    x0 = x[:, 0].astype(jnp.int32)
    x1 = x[:, 1].astype(jnp.int32)
    dep = depth.astype(jnp.int32)
    e1p = jnp.pad(emb1, ((0, _V1P - emb1.shape[0]),
                         (_D0, _EMB_DIM - _D0 - _D1)))
    e0f = emb0.reshape(-1)
    edf = depth_table.reshape(-1)

    mesh = plsc.VectorSubcoreMesh(core_axis_name="c", subcore_axis_name="s")
    nidx = _RANGE + _TAIL + _TAIL2

    @functools.partial(
        pl.kernel,
        mesh=mesh,
        compiler_params=pltpu.CompilerParams(needs_layout_passes=False),
        out_type=jax.ShapeDtypeStruct((_N, _EMB_DIM), jnp.float32),
        scratch_types=[
            pltpu.VMEM((nidx,), jnp.int32),
            pltpu.VMEM((nidx,), jnp.int32),
            pltpu.VMEM((nidx,), jnp.int32),
            pltpu.VMEM((_V0 * _D0,), jnp.float32),
            pltpu.VMEM((_VD * _DD,), jnp.float32),
            pltpu.VMEM((_B, _EMB_DIM), jnp.float32),
            pltpu.VMEM((_B, _EMB_DIM), jnp.float32),
            pltpu.VMEM_SHARED((_V1P, _EMB_DIM), jnp.float32),
            pltpu.SemaphoreType.DMA,
            pltpu.SemaphoreType.DMA,
            pltpu.SemaphoreType.DMA,
        ],
    )
    def run(x0_hbm, x1_hbm, dep_hbm, e0_hbm, e1_hbm, ed_hbm, out_hbm,
            idx0, idx1, idxd, e0_res, ed_res, rows_a, rows_b, e1_sh,
            gsem, wsem, isem):
        wid = lax.axis_index("s") * 2 + lax.axis_index("c")
        sid = lax.axis_index("s")
        base = wid * _RANGE
        lanes = lax.iota(jnp.int32, 16)

        pltpu.sync_copy(e0_hbm, e0_res)
        pltpu.sync_copy(ed_hbm, ed_res)
        tbase = _TAILBASE + wid * _TAIL
        stage = [
            pltpu.async_copy(e1_hbm.at[pl.ds(sid * _SLAB, _SLAB)],
                             e1_sh.at[pl.ds(sid * _SLAB, _SLAB)], isem),
            pltpu.async_copy(x0_hbm.at[pl.ds(base, _RANGE)],
                             idx0.at[pl.ds(0, _RANGE)], isem),
            pltpu.async_copy(x1_hbm.at[pl.ds(base, _RANGE)],
                             idx1.at[pl.ds(0, _RANGE)], isem),
            pltpu.async_copy(dep_hbm.at[pl.ds(base, _RANGE)],
                             idxd.at[pl.ds(0, _RANGE)], isem),
            pltpu.async_copy(x0_hbm.at[pl.ds(tbase, _TAIL)],
                             idx0.at[pl.ds(_RANGE, _TAIL)], isem),
            pltpu.async_copy(x1_hbm.at[pl.ds(tbase, _TAIL)],
                             idx1.at[pl.ds(_RANGE, _TAIL)], isem),
            pltpu.async_copy(dep_hbm.at[pl.ds(tbase, _TAIL)],
                             idxd.at[pl.ds(_RANGE, _TAIL)], isem),
        ]
        for s in stage:
            s.wait()
        plsc.subcore_barrier()

        def fire_gather(ioff, nrows, buf):
            return pltpu.async_copy(
                e1_sh.at[idx1.at[pl.ds(ioff, nrows)]],
                buf.at[pl.ds(0, nrows)], gsem)

        def wait_gather(nrows, buf):
            pltpu.make_async_copy(
                e1_hbm.at[pl.ds(0, nrows)], buf.at[pl.ds(0, nrows)], gsem).wait()

        def fill(ioff, nrows, buf):
            @plsc.parallel_loop(0, nrows // 16, unroll=1)
            def grp(t):
                r0 = t * 16
                a = idx0[pl.ds(ioff + r0, 16)] * _D0
                d = jnp.minimum(idxd[pl.ds(ioff + r0, 16)], _MAX_DEPTH) * _DD
                for l in range(16):
                    r = r0 + l
                    i0 = a[l]
                    idp = d[l]
                    buf[r, pl.ds(0, 16)] = e0_res[pl.ds(i0, 16)]
                    buf[r, pl.ds(16, 16)] = e0_res[pl.ds(i0 + 16, 16)]
                    buf[r, pl.ds(_D0 - 16, 16)] = e0_res[pl.ds(i0 + _D0 - 16, 16)]
                    buf[r, pl.ds(_D0 + _D1, 16)] = ed_res[pl.ds(idp, 16)]
                    buf[r, pl.ds(_D0 + _D1 + 16, 16)] = ed_res[pl.ds(idp + 16, 16)]
                    buf[r, pl.ds(_EMB_DIM - 16, 16)] = ed_res[pl.ds(idp + _DD - 16, 16)]

        def fire_write(obase, nrows, buf):
            return pltpu.async_copy(
                buf.at[pl.ds(0, nrows)], out_hbm.at[pl.ds(obase, nrows)], wsem)

        def wait_write(nrows, buf):
            pltpu.make_async_copy(
                buf.at[pl.ds(0, nrows)], out_hbm.at[pl.ds(0, nrows)], wsem).wait()

        fire_gather(0, _B, rows_a)
        fire_gather(_B, _B, rows_b)

        def pair(p, _):
            ca = 2 * p * _B
            cb = ca + _B
            wait_gather(_B, rows_a)
            fill(ca, _B, rows_a)
            fire_write(base + ca, _B, rows_a)
            wait_gather(_B, rows_b)
            fill(cb, _B, rows_b)
            fire_write(base + cb, _B, rows_b)

            @pl.when(p < _PAIRS - 1)
            def _():
                wait_write(_B, rows_a)
                fire_gather(cb + _B, _B, rows_a)
                wait_write(_B, rows_b)
                fire_gather(cb + 2 * _B, _B, rows_b)
            return 0
        lax.fori_loop(0, _PAIRS, pair, 0)

        wait_write(_B, rows_a)
        fire_gather(_RANGE, _TAIL, rows_a)
        wait_gather(_TAIL, rows_a)
        fill(_RANGE, _TAIL, rows_a)
        tdesc = fire_write(tbase, _TAIL, rows_a)
        wait_write(_B, rows_b)
        tdesc.wait()

        @pl.when(wid == _NW - 1)
        def _():
            t2base = _TAILBASE + _NW * _TAIL
            s2 = [pltpu.async_copy(x0_hbm.at[pl.ds(t2base, _TAIL2)],
                                   idx0.at[pl.ds(_RANGE + _TAIL, _TAIL2)], isem),
                  pltpu.async_copy(x1_hbm.at[pl.ds(t2base, _TAIL2)],
                                   idx1.at[pl.ds(_RANGE + _TAIL, _TAIL2)], isem),
                  pltpu.async_copy(dep_hbm.at[pl.ds(t2base, _TAIL2)],
                                   idxd.at[pl.ds(_RANGE + _TAIL, _TAIL2)], isem)]
            for s in s2:
                s.wait()
            for off, cnt in ((0, _B), (_B, _TAIL2 - _B)):
                fire_gather(_RANGE + _TAIL + off, cnt, rows_b)
                wait_gather(cnt, rows_b)
                fill(_RANGE + _TAIL + off, cnt, rows_b)
                fire_write(t2base + off, cnt, rows_b).wait()

    return run(x0, x1, dep, e0f, e1p, edf)
